# baseline (speedup 1.0000x reference)
import functools

import jax
import jax.numpy as jnp
from jax import lax
from jax.experimental import pallas as pl
from jax.experimental.pallas import tpu as pltpu
from jax.experimental.pallas import tpu_sc as plsc

_B = 1024
_T = 12
_NT = _B * _T
_NS = 2048
_DA = 256
_DI = 128
_DM = 1024
_DG = _DA + _DI + _DM

_NC = 2
_NSUB = 16
_NW = _NC * _NSUB
_PER_W = _NT // _NW
_CHUNK = 32
_NCHUNK = _PER_W // _CHUNK


def _sc_gather_body(tok_hbm, wg_hbm, gg_hbm,
                    idx_v, g0, g1, s0, s1):
    wid = lax.axis_index("s") * _NC + lax.axis_index("c")
    base = wid * _PER_W
    pltpu.sync_copy(tok_hbm.at[pl.ds(base, _PER_W)], idx_v)
    for k in range(_PER_W // 16):
        v = idx_v[pl.ds(k * 16, 16)]
        idx_v[pl.ds(k * 16, 16)] = jnp.maximum(v - 1, 0)
    bufs = (g0, g1)
    sems = (s0, s1)

    def start(c):
        ix = idx_v.at[pl.ds(c * _CHUNK, _CHUNK)]
        return pltpu.async_copy(wg_hbm.at[ix], bufs[c % 2], sems[c % 2])

    inflight = start(0)
    for c in range(_NCHUNK):
        inflight.wait()
        inflight = start(c + 1) if c + 1 < _NCHUNK else None
        off = base + c * _CHUNK
        pltpu.sync_copy(bufs[c % 2], gg_hbm.at[pl.ds(off, _CHUNK)])


@functools.cache
def _make_sc_gather():
    return pl.kernel(
        _sc_gather_body,
        out_type=jax.ShapeDtypeStruct((_NT, _DG), jnp.float32),
        mesh=plsc.VectorSubcoreMesh(core_axis_name="c", subcore_axis_name="s",
                                    num_cores=_NC, num_subcores=_NSUB),
        scratch_types=(
            pltpu.VMEM((_PER_W,), jnp.int32),
            pltpu.VMEM((_CHUNK, _DG), jnp.float32),
            pltpu.VMEM((_CHUNK, _DG), jnp.float32),
            pltpu.SemaphoreType.DMA,
            pltpu.SemaphoreType.DMA,
        ),
    )


def _colsum_body(wg_ref, cg_out):
    cg_out[...] = wg_ref[...].sum(axis=1, keepdims=True)


def _colsums(wg):
    return pl.pallas_call(
        _colsum_body,
        out_shape=jax.ShapeDtypeStruct((1, 1, _DG), jnp.float32),
    )(wg)


_BBLK = 32


def _tc_body(st_ref, at_ref, it_ref, mt_ref, gg_ref, cg_ref,
             sp_out, ab_out, im_out, mv_out):
    st = st_ref[...]
    known = st > 0
    knownf = known.astype(jnp.float32)

    iota_s = lax.broadcasted_iota(jnp.int32, (_T, _BBLK, _NS), 2)
    oh = (iota_s == st - 1).astype(jnp.float32)
    counts = oh.sum(axis=0, keepdims=True)
    kcnt = knownf.sum(axis=0, keepdims=True)
    t_unk = jnp.maximum(float(_NS) - kcnt, 1.0)
    u = (1.0 - counts) / t_unk
    sp_out[...] = jnp.where(known, oh, u)

    def prior(lo, d):
        g = gg_ref[:, :, lo:lo + d]
        team = (g * knownf).sum(axis=0, keepdims=True)
        unk = (cg_ref[:, :, lo:lo + d] - team) / t_unk
        return jnp.where(known, g, unk)

    raw_a = prior(0, _DA)
    raw_a = raw_a / jnp.maximum(raw_a.sum(-1, keepdims=True), 1.0)
    at = at_ref[...]
    iota_a = lax.broadcasted_iota(jnp.int32, (_T, _BBLK, _DA), 2)
    a_oh = (iota_a == at - 1).astype(jnp.float32)
    ab_out[...] = jnp.where(at > 0, a_oh, raw_a)

    raw_i = prior(_DA, _DI)
    raw_i = raw_i / jnp.maximum(raw_i.sum(-1, keepdims=True), 1.0)
    it = it_ref[...]
    iota_i = lax.broadcasted_iota(jnp.int32, (_T, _BBLK, _DI), 2)
    i_oh = (iota_i == it - 1).astype(jnp.float32)
    im_out[...] = jnp.where(it > 0, i_oh, raw_i)

    all_unk = prior(_DA + _DI, _DM)
    mt = mt_ref[...]
    iota_m = lax.broadcasted_iota(jnp.int32, (_T, _BBLK, _DM), 2)
    m_known = jnp.zeros((_T, _BBLK, _DM), jnp.float32)
    for k in range(4):
        mk = mt[..., k:k + 1]
        m_known = m_known + (iota_m == mk - 1).astype(jnp.float32)
    m_unk = all_unk - m_known
    m_unk = m_unk / jnp.maximum(m_unk.sum(-1, keepdims=True), 1.0)
    num_missing = 4.0 - (m_known > 0).sum(-1, keepdims=True).astype(jnp.float32)
    move_mask = known & (mt.sum(-1, keepdims=True) != 0)
    mv_out[...] = jnp.where(move_mask,
                            m_known + num_missing * m_unk, 4.0 * m_unk)


def _tc_epilogue(st, at, it, mt, gg, cg):
    grid = (_B // _BBLK,)
    tok3 = lambda i: (0, i, 0)
    full = lambda i: (0, 0, 0)
    return pl.pallas_call(
        _tc_body,
        grid=grid,
        in_specs=[
            pl.BlockSpec((_T, _BBLK, 1), tok3),
            pl.BlockSpec((_T, _BBLK, 1), tok3),
            pl.BlockSpec((_T, _BBLK, 1), tok3),
            pl.BlockSpec((_T, _BBLK, 4), tok3),
            pl.BlockSpec((_T, _BBLK, _DG), tok3),
            pl.BlockSpec((1, 1, _DG), full),
        ],
        out_specs=[
            pl.BlockSpec((_T, _BBLK, _NS), tok3),
            pl.BlockSpec((_T, _BBLK, _DA), tok3),
            pl.BlockSpec((_T, _BBLK, _DI), tok3),
            pl.BlockSpec((_T, _BBLK, _DM), tok3),
        ],
        out_shape=[
            jax.ShapeDtypeStruct((_T, _B, _NS), jnp.float32),
            jax.ShapeDtypeStruct((_T, _B, _DA), jnp.float32),
            jax.ShapeDtypeStruct((_T, _B, _DI), jnp.float32),
            jax.ShapeDtypeStruct((_T, _B, _DM), jnp.float32),
        ],
    )(st, at, it, mt, gg, cg)


def kernel(species_token, ability_token, item_token, move_tokens,
           species_onehot_w, all_abilities_w, abilities_onehot_w,
           all_items_w, items_onehot_w, all_moves_w, moves_onehot_w):
    st_t = species_token.T.astype(jnp.int32)
    at_t = ability_token.T.astype(jnp.int32)
    it_t = item_token.T.astype(jnp.int32)
    mt_t = jnp.transpose(move_tokens, (1, 0, 2)).astype(jnp.int32)
    wg = jnp.concatenate([all_abilities_w, all_items_w, all_moves_w], axis=1)
    gg = _make_sc_gather()(st_t.reshape(-1), wg)
    cg = _colsums(wg.reshape(1, _NS, _DG))
    sp, ab, im, mv = _tc_epilogue(
        st_t.reshape(_T, _B, 1), at_t.reshape(_T, _B, 1),
        it_t.reshape(_T, _B, 1), mt_t,
        gg.reshape(_T, _B, _DG), cg)
    tr = lambda x: jnp.transpose(x, (1, 0, 2))
    return tr(sp), tr(ab), tr(im), tr(mv)


# speedup vs baseline: 1.1674x; 1.1674x over previous
import functools

import jax
import jax.numpy as jnp
from jax import lax
from jax.experimental import pallas as pl
from jax.experimental.pallas import tpu as pltpu
from jax.experimental.pallas import tpu_sc as plsc

_B = 1024
_T = 12
_NT = _B * _T
_NS = 2048
_DA = 256
_DI = 128
_DM = 1024
_DG = _DA + _DI + _DM
_DGP = 1536
_DH = _DGP // 2

_NC = 2
_NSUB = 16
_NW = _NC * _NSUB
_PER_W = _NT // _NW
_CHUNK = 32
_NCHUNK = _PER_W // _CHUNK


def _sc_gather_body(tok_hbm, wg_hbm, gg_hbm,
                    idx_v, g0, g1, s0, s1):
    wid = lax.axis_index("s") * _NC + lax.axis_index("c")
    base = wid * _PER_W
    pltpu.sync_copy(tok_hbm.at[pl.ds(base, _PER_W)], idx_v)
    for k in range(_PER_W // 16):
        v = idx_v[pl.ds(k * 16, 16)]
        idx_v[pl.ds(k * 16, 16)] = jnp.maximum(v - 1, 0)
    bufs = (g0, g1)
    sems = (s0, s1)

    def start(c):
        ix = idx_v.at[pl.ds(c * _CHUNK, _CHUNK)]
        return pltpu.async_copy(wg_hbm.at[ix], bufs[c % 2], sems[c % 2])

    inflight = start(0)
    for c in range(_NCHUNK):
        inflight.wait()
        inflight = start(c + 1) if c + 1 < _NCHUNK else None
        off = base + c * _CHUNK
        pltpu.sync_copy(bufs[c % 2], gg_hbm.at[pl.ds(off, _CHUNK)])


@functools.cache
def _make_sc_gather():
    return pl.kernel(
        _sc_gather_body,
        out_type=jax.ShapeDtypeStruct((_NT, _DH), jnp.int32),
        mesh=plsc.VectorSubcoreMesh(core_axis_name="c", subcore_axis_name="s",
                                    num_cores=_NC, num_subcores=_NSUB),
        scratch_types=(
            pltpu.VMEM((_PER_W,), jnp.int32),
            pltpu.VMEM((_CHUNK, _DH), jnp.int32),
            pltpu.VMEM((_CHUNK, _DH), jnp.int32),
            pltpu.SemaphoreType.DMA,
            pltpu.SemaphoreType.DMA,
        ),
    )


def _colsum_body(wa_ref, wi_ref, wm_ref, cg_out):
    cg_out[...] = jnp.concatenate(
        [wa_ref[...].sum(axis=1, keepdims=True),
         wi_ref[...].sum(axis=1, keepdims=True),
         wm_ref[...].sum(axis=1, keepdims=True)], axis=2)


def _colsums(wa, wi, wm):
    return pl.pallas_call(
        _colsum_body,
        out_shape=jax.ShapeDtypeStruct((1, 1, _DG), jnp.float32),
    )(wa, wi, wm)


_BBLK = 32


def _tc_body(st_ref, at_ref, it_ref, mt_ref, gg_ref, cg_ref,
             sp_out, ab_out, im_out, mv_out):
    st = st_ref[...]
    known = st > 0
    knownf = known.astype(jnp.float32)

    iota_s = lax.broadcasted_iota(jnp.int32, (_T, _BBLK, _NS), 2)
    oh = (iota_s == st - 1).astype(jnp.float32)
    counts = oh.sum(axis=0, keepdims=True)
    kcnt = knownf.sum(axis=0, keepdims=True)
    t_unk = jnp.maximum(float(_NS) - kcnt, 1.0)
    u = (1.0 - counts) / t_unk
    sp_out[...] = jnp.where(known, oh, u)

    gg = gg_ref[...]
    g_lo = lax.bitcast_convert_type(lax.shift_left(gg, 16), jnp.float32)
    g_hi = lax.bitcast_convert_type(
        jnp.bitwise_and(gg, jnp.int32(-65536)), jnp.float32)

    def prior(g, lo, d):
        team = (g * knownf).sum(axis=0, keepdims=True)
        unk = (cg_ref[:, :, lo:lo + d] - team) / t_unk
        return jnp.where(known, g, unk)

    raw_a = prior(g_lo[:, :, 0:_DA], 0, _DA)
    raw_a = raw_a / jnp.maximum(raw_a.sum(-1, keepdims=True), 1.0)
    at = at_ref[...]
    iota_a = lax.broadcasted_iota(jnp.int32, (_T, _BBLK, _DA), 2)
    a_oh = (iota_a == at - 1).astype(jnp.float32)
    ab_out[...] = jnp.where(at > 0, a_oh, raw_a)

    raw_i = prior(g_lo[:, :, _DA:_DA + _DI], _DA, _DI)
    raw_i = raw_i / jnp.maximum(raw_i.sum(-1, keepdims=True), 1.0)
    it = it_ref[...]
    iota_i = lax.broadcasted_iota(jnp.int32, (_T, _BBLK, _DI), 2)
    i_oh = (iota_i == it - 1).astype(jnp.float32)
    im_out[...] = jnp.where(it > 0, i_oh, raw_i)

    g_m = jnp.concatenate(
        [g_lo[:, :, _DA + _DI:_DH], g_hi[:, :, 0:_DM - (_DH - _DA - _DI)]],
        axis=2)
    all_unk = prior(g_m, _DA + _DI, _DM)
    mt = mt_ref[...]
    iota_m = lax.broadcasted_iota(jnp.int32, (_T, _BBLK, _DM), 2)
    m_known = jnp.zeros((_T, _BBLK, _DM), jnp.float32)
    for k in range(4):
        mk = mt[..., k:k + 1]
        m_known = m_known + (iota_m == mk - 1).astype(jnp.float32)
    m_unk = all_unk - m_known
    m_unk = m_unk / jnp.maximum(m_unk.sum(-1, keepdims=True), 1.0)
    num_missing = 4.0 - (m_known > 0).sum(-1, keepdims=True).astype(jnp.float32)
    move_mask = known & (mt.sum(-1, keepdims=True) != 0)
    mv_out[...] = jnp.where(move_mask,
                            m_known + num_missing * m_unk, 4.0 * m_unk)


def _tc_epilogue(st, at, it, mt, gg, cg):
    grid = (_B // _BBLK,)
    tok3 = lambda i: (0, i, 0)
    full = lambda i: (0, 0, 0)
    return pl.pallas_call(
        _tc_body,
        grid=grid,
        in_specs=[
            pl.BlockSpec((_T, _BBLK, 1), tok3),
            pl.BlockSpec((_T, _BBLK, 1), tok3),
            pl.BlockSpec((_T, _BBLK, 1), tok3),
            pl.BlockSpec((_T, _BBLK, 4), tok3),
            pl.BlockSpec((_T, _BBLK, _DH), tok3),
            pl.BlockSpec((1, 1, _DG), full),
        ],
        out_specs=[
            pl.BlockSpec((_T, _BBLK, _NS), tok3),
            pl.BlockSpec((_T, _BBLK, _DA), tok3),
            pl.BlockSpec((_T, _BBLK, _DI), tok3),
            pl.BlockSpec((_T, _BBLK, _DM), tok3),
        ],
        out_shape=[
            jax.ShapeDtypeStruct((_T, _B, _NS), jnp.float32),
            jax.ShapeDtypeStruct((_T, _B, _DA), jnp.float32),
            jax.ShapeDtypeStruct((_T, _B, _DI), jnp.float32),
            jax.ShapeDtypeStruct((_T, _B, _DM), jnp.float32),
        ],
    )(st, at, it, mt, gg, cg)


def kernel(species_token, ability_token, item_token, move_tokens,
           species_onehot_w, all_abilities_w, abilities_onehot_w,
           all_items_w, items_onehot_w, all_moves_w, moves_onehot_w):
    st_t = species_token.T.astype(jnp.int32)
    at_t = ability_token.T.astype(jnp.int32)
    it_t = item_token.T.astype(jnp.int32)
    mt_t = jnp.transpose(move_tokens, (1, 0, 2)).astype(jnp.int32)
    wgp = jnp.concatenate(
        [all_abilities_w, all_items_w, all_moves_w,
         jnp.zeros((_NS, _DGP - _DG), jnp.float32)],
        axis=1).astype(jnp.bfloat16)
    wg = lax.bitcast_convert_type(
        jnp.stack([wgp[:, :_DH], wgp[:, _DH:]], axis=-1),
        jnp.int32)
    gg = _make_sc_gather()(st_t.reshape(-1), wg)
    cg = _colsums(all_abilities_w.reshape(1, _NS, _DA),
                  all_items_w.reshape(1, _NS, _DI),
                  all_moves_w.reshape(1, _NS, _DM))
    sp, ab, im, mv = _tc_epilogue(
        st_t.reshape(_T, _B, 1), at_t.reshape(_T, _B, 1),
        it_t.reshape(_T, _B, 1), mt_t,
        gg.reshape(_T, _B, _DH), cg)
    tr = lambda x: jnp.transpose(x, (1, 0, 2))
    return tr(sp), tr(ab), tr(im), tr(mv)
